# Initial kernel scaffold; baseline (speedup 1.0000x reference)
import jax
import jax.numpy as jnp
from jax.experimental import pallas as pl


def kernel(outputs, W):
    raise NotImplementedError("write your pallas kernel here")


# speedup vs baseline: 10.7611x; 10.7611x over previous
import functools

import jax
import jax.numpy as jnp
from jax import lax
from jax.experimental import pallas as pl
from jax.experimental.pallas import tpu as pltpu
from jax.experimental.pallas import tpu_sc as plsc

N = 8192
M = 126
SIZE = M + 2
NC, NS, L = 2, 16, 16
NW = NC * NS
RPW = N // NW
G = 2
NG = RPW // G

_NEG_INF = float("-inf")
_BIG = 1 << 30


def _splat(v):
    return jnp.full((L,), v)


def _sc_body(srco, dsto, uavo, wflat, out_hbm,
             w_v, uav_v, uavp_v, srco_v, srcp_v, dsto_v, dstp_v,
             mask_v, md2_v):
    wid = lax.axis_index("s") * NC + lax.axis_index("c")
    base_row = wid * RPW

    pltpu.sync_copy(wflat, w_v)
    pltpu.sync_copy(uavo, uav_v)
    for coord in range(3):
        pltpu.sync_copy(srco.at[pl.ds(coord * N + base_row, RPW)],
                        srco_v.at[pl.ds(coord * RPW, RPW)])
        pltpu.sync_copy(dsto.at[pl.ds(coord * N + base_row, RPW)],
                        dsto_v.at[pl.ds(coord * RPW, RPW)])

    iota = lax.iota(jnp.int32, L)
    w = [[w_v[pl.ds((3 * i + j) * L, L)] for j in range(3)] for i in range(3)]

    for c in range(SIZE // L):
        ux = uav_v[pl.ds(c * L, L)]
        uy = uav_v[pl.ds(SIZE + c * L, L)]
        uz = uav_v[pl.ds(2 * SIZE + c * L, L)]
        for j in range(3):
            uavp_v[pl.ds(j * SIZE + c * L, L)] = (
                ux * w[0][j] + uy * w[1][j] + uz * w[2][j])
    for c in range(RPW // L):
        sx = srco_v[pl.ds(c * L, L)]
        sy = srco_v[pl.ds(RPW + c * L, L)]
        sz = srco_v[pl.ds(2 * RPW + c * L, L)]
        dx = dsto_v[pl.ds(c * L, L)]
        dy = dsto_v[pl.ds(RPW + c * L, L)]
        dz = dsto_v[pl.ds(2 * RPW + c * L, L)]
        for j in range(3):
            srcp_v[pl.ds(j * RPW + c * L, L)] = (
                sx * w[0][j] + sy * w[1][j] + sz * w[2][j])
            dstp_v[pl.ds(j * RPW + c * L, L)] = (
                dx * w[0][j] + dy * w[1][j] + dz * w[2][j])

    neginf_v = jnp.full((L,), _NEG_INF, jnp.float32)
    zero_v = jnp.zeros((L,), jnp.float32)
    mask0 = jnp.where(iota == 0, _NEG_INF, 0.0).astype(jnp.float32)
    lane15 = iota == (L - 1)
    lane0 = iota == 0

    def group_body(g, _):
        rows = [g * G + r for r in range(G)]
        rowv = [jnp.full((L,), rows[r], jnp.int32) for r in range(G)]
        dox = [plsc.load_gather(dsto_v, [rowv[r]]) for r in range(G)]
        doy = [plsc.load_gather(dsto_v, [rowv[r] + RPW]) for r in range(G)]
        doz = [plsc.load_gather(dsto_v, [rowv[r] + 2 * RPW]) for r in range(G)]
        dpx = [plsc.load_gather(dstp_v, [rowv[r]]) for r in range(G)]
        dpy = [plsc.load_gather(dstp_v, [rowv[r] + RPW]) for r in range(G)]
        dpz = [plsc.load_gather(dstp_v, [rowv[r] + 2 * RPW]) for r in range(G)]
        ox0 = [plsc.load_gather(srco_v, [rowv[r]]) for r in range(G)]
        oy0 = [plsc.load_gather(srco_v, [rowv[r] + RPW]) for r in range(G)]
        oz0 = [plsc.load_gather(srco_v, [rowv[r] + 2 * RPW]) for r in range(G)]
        px0 = [plsc.load_gather(srcp_v, [rowv[r]]) for r in range(G)]
        py0 = [plsc.load_gather(srcp_v, [rowv[r] + RPW]) for r in range(G)]
        pz0 = [plsc.load_gather(srcp_v, [rowv[r] + 2 * RPW]) for r in range(G)]
        for r in range(G):
            mask_v[r, pl.ds(0, L)] = mask0
            for c in range(1, SIZE // L):
                mask_v[r, pl.ds(c * L, L)] = zero_v

        def step(k, st):
            xp, yp, zp, xo, yo, zo, md2 = st
            m127 = jnp.where(k == 0, _NEG_INF, jnp.float32(0.0))
            cand = [(uavp_v[pl.ds(c * L, L)],
                     uavp_v[pl.ds(SIZE + c * L, L)],
                     uavp_v[pl.ds(2 * SIZE + c * L, L)])
                    for c in range(SIZE // L)]
            nxp, nyp, nzp = list(xp), list(yp), list(zp)
            nxo, nyo, nzo = list(xo), list(yo), list(zo)
            nmd = list(md2)
            for r in range(G):
                bestv = neginf_v
                besti = jnp.full((L,), _BIG, jnp.int32)
                for c in range(SIZE // L):
                    cx, cy, cz = cand[c]
                    mk = mask_v[r, pl.ds(c * L, L)]
                    ddx = cx - xp[r]
                    ddy = cy - yp[r]
                    ddz = cz - zp[r]
                    d2 = ddx * ddx + ddy * ddy + ddz * ddz
                    s = mk - d2
                    if c == SIZE // L - 1:
                        fx = dpx[r] - xp[r]
                        fy = dpy[r] - yp[r]
                        fz = dpz[r] - zp[r]
                        d2d = fx * fx + fy * fy + fz * fz
                        s = jnp.where(lane15, m127 - d2d, s)
                    gt = s > bestv
                    bestv = jnp.where(gt, s, bestv)
                    besti = jnp.where(gt, iota + (c * L), besti)
                mval = jnp.max(bestv)
                idx = jnp.min(jnp.where(bestv == mval, besti, _BIG))
                idxv = jnp.full((L,), idx, jnp.int32)
                is_dst = idx == jnp.int32(SIZE - 1)
                gx = plsc.load_gather(uavp_v, [idxv])
                gy = plsc.load_gather(uavp_v, [idxv + SIZE])
                gz = plsc.load_gather(uavp_v, [idxv + 2 * SIZE])
                hx = plsc.load_gather(uav_v, [idxv])
                hy = plsc.load_gather(uav_v, [idxv + SIZE])
                hz = plsc.load_gather(uav_v, [idxv + 2 * SIZE])
                nxp[r] = jnp.where(is_dst, dpx[r], gx)
                nyp[r] = jnp.where(is_dst, dpy[r], gy)
                nzp[r] = jnp.where(is_dst, dpz[r], gz)
                nxo[r] = jnp.where(is_dst, dox[r], hx)
                nyo[r] = jnp.where(is_dst, doy[r], hy)
                nzo[r] = jnp.where(is_dst, doz[r], hz)
                ex = nxo[r] - xo[r]
                ey = nyo[r] - yo[r]
                ez = nzo[r] - zo[r]
                nmd[r] = jnp.maximum(md2[r], ex * ex + ey * ey + ez * ez)
                plsc.store_scatter(mask_v.at[r], [idxv], neginf_v,
                                   mask=lane0 & (idxv != SIZE - 1))
            return (tuple(nxp), tuple(nyp), tuple(nzp),
                    tuple(nxo), tuple(nyo), tuple(nzo), tuple(nmd))

        init = (tuple(px0), tuple(py0), tuple(pz0),
                tuple(ox0), tuple(oy0), tuple(oz0),
                tuple(zero_v for _ in range(G)))
        st = lax.fori_loop(0, SIZE, step, init)
        md2f = st[6]
        for r in range(G):
            plsc.store_scatter(md2_v, [rowv[r]], md2f[r], mask=lane0)
        return 0

    lax.fori_loop(0, NG, group_body, 0)
    pltpu.sync_copy(md2_v, out_hbm.at[pl.ds(base_row, RPW)])


_sc_path = functools.partial(
    pl.kernel,
    out_type=jax.ShapeDtypeStruct((N,), jnp.float32),
    mesh=plsc.VectorSubcoreMesh(core_axis_name="c", subcore_axis_name="s"),
    compiler_params=pltpu.CompilerParams(needs_layout_passes=False),
    scratch_types=[
        pltpu.VMEM((9 * L,), jnp.float32),
        pltpu.VMEM((3 * SIZE,), jnp.float32),
        pltpu.VMEM((3 * SIZE,), jnp.float32),
        pltpu.VMEM((3 * RPW,), jnp.float32),
        pltpu.VMEM((3 * RPW,), jnp.float32),
        pltpu.VMEM((3 * RPW,), jnp.float32),
        pltpu.VMEM((3 * RPW,), jnp.float32),
        pltpu.VMEM((G, SIZE), jnp.float32),
        pltpu.VMEM((RPW,), jnp.float32),
    ],
)(_sc_body)


def _mean_sqrt_body(x_ref, o_ref):
    o_ref[0, 0] = jnp.sum(jnp.sqrt(x_ref[...])) * jnp.float32(1.0 / N)


_mean_sqrt = pl.pallas_call(
    _mean_sqrt_body,
    out_shape=jax.ShapeDtypeStruct((1, 1), jnp.float32),
    out_specs=pl.BlockSpec(memory_space=pltpu.SMEM),
)


def kernel(outputs, W):
    src = outputs[:N]
    dst = outputs[N:2 * N]
    uav = outputs[2 * N:]
    srco = src.T.reshape(-1)
    dsto = dst.T.reshape(-1)
    uavo = jnp.zeros((3, SIZE), jnp.float32).at[:, 1:SIZE - 1].set(uav.T).reshape(-1)
    wflat = jnp.repeat(W.reshape(-1), L)
    md2 = _sc_path(srco, dsto, uavo, wflat)
    return _mean_sqrt(md2.reshape(N // 128, 128))[0, 0]


# speedup vs baseline: 14.9878x; 1.3928x over previous
import functools

import jax
import jax.numpy as jnp
from jax import lax
from jax.experimental import pallas as pl
from jax.experimental.pallas import tpu as pltpu
from jax.experimental.pallas import tpu_sc as plsc

N = 8192
M = 126
SIZE = M + 2
NC, NS, L = 2, 16, 16
NW = NC * NS
RPW = N // NW
G = 4
NG = RPW // G
NCH = SIZE // L

_NEG_INF = float("-inf")
_BIG = 1 << 30


def _sc_body(srco, dsto, uavo, wflat, out_hbm,
             w_v, uav_v, uavp_v, uavp2_v, nm_v,
             srco_v, srcp_v, dsto_v, dstp_v,
             nmm_v, md2_v):
    wid = lax.axis_index("s") * NC + lax.axis_index("c")
    base_row = wid * RPW

    pltpu.sync_copy(wflat, w_v)
    pltpu.sync_copy(uavo, uav_v)
    for coord in range(3):
        pltpu.sync_copy(srco.at[pl.ds(coord * N + base_row, RPW)],
                        srco_v.at[pl.ds(coord * RPW, RPW)])
        pltpu.sync_copy(dsto.at[pl.ds(coord * N + base_row, RPW)],
                        dsto_v.at[pl.ds(coord * RPW, RPW)])

    iota = lax.iota(jnp.int32, L)
    w = [[w_v[pl.ds((3 * i + j) * L, L)] for j in range(3)] for i in range(3)]

    neginf_v = jnp.full((L,), _NEG_INF, jnp.float32)
    zero_v = jnp.zeros((L,), jnp.float32)
    lane15 = iota == (L - 1)
    lane0 = iota == 0

    for c in range(NCH):
        ux = uav_v[pl.ds(c * L, L)]
        uy = uav_v[pl.ds(SIZE + c * L, L)]
        uz = uav_v[pl.ds(2 * SIZE + c * L, L)]
        pj = [ux * w[0][j] + uy * w[1][j] + uz * w[2][j] for j in range(3)]
        for j in range(3):
            uavp_v[pl.ds(j * SIZE + c * L, L)] = pj[j]
            uavp2_v[pl.ds(j * SIZE + c * L, L)] = pj[j] + pj[j]
        nm = -(pj[0] * pj[0] + pj[1] * pj[1] + pj[2] * pj[2])
        if c == 0:
            nm = jnp.where(lane0, _NEG_INF, nm)
        nm_v[pl.ds(c * L, L)] = nm
    for c in range(RPW // L):
        sx = srco_v[pl.ds(c * L, L)]
        sy = srco_v[pl.ds(RPW + c * L, L)]
        sz = srco_v[pl.ds(2 * RPW + c * L, L)]
        dx = dsto_v[pl.ds(c * L, L)]
        dy = dsto_v[pl.ds(RPW + c * L, L)]
        dz = dsto_v[pl.ds(2 * RPW + c * L, L)]
        for j in range(3):
            srcp_v[pl.ds(j * RPW + c * L, L)] = (
                sx * w[0][j] + sy * w[1][j] + sz * w[2][j])
            dstp_v[pl.ds(j * RPW + c * L, L)] = (
                dx * w[0][j] + dy * w[1][j] + dz * w[2][j])

    def group_body(g, _):
        rows = [g * G + r for r in range(G)]
        rowv = [jnp.full((L,), rows[r], jnp.int32) for r in range(G)]
        dox = [plsc.load_gather(dsto_v, [rowv[r]]) for r in range(G)]
        doy = [plsc.load_gather(dsto_v, [rowv[r] + RPW]) for r in range(G)]
        doz = [plsc.load_gather(dsto_v, [rowv[r] + 2 * RPW]) for r in range(G)]
        dgx = [plsc.load_gather(dstp_v, [rowv[r]]) for r in range(G)]
        dgy = [plsc.load_gather(dstp_v, [rowv[r] + RPW]) for r in range(G)]
        dgz = [plsc.load_gather(dstp_v, [rowv[r] + 2 * RPW]) for r in range(G)]
        dpx2 = [dgx[r] + dgx[r] for r in range(G)]
        dpy2 = [dgy[r] + dgy[r] for r in range(G)]
        dpz2 = [dgz[r] + dgz[r] for r in range(G)]
        dnorm = [-(dgx[r] * dgx[r] + dgy[r] * dgy[r] + dgz[r] * dgz[r])
                 for r in range(G)]
        ox0 = [plsc.load_gather(srco_v, [rowv[r]]) for r in range(G)]
        oy0 = [plsc.load_gather(srco_v, [rowv[r] + RPW]) for r in range(G)]
        oz0 = [plsc.load_gather(srco_v, [rowv[r] + 2 * RPW]) for r in range(G)]
        px0 = [plsc.load_gather(srcp_v, [rowv[r]]) for r in range(G)]
        py0 = [plsc.load_gather(srcp_v, [rowv[r] + RPW]) for r in range(G)]
        pz0 = [plsc.load_gather(srcp_v, [rowv[r] + 2 * RPW]) for r in range(G)]
        for c in range(NCH):
            nmc = nm_v[pl.ds(c * L, L)]
            for r in range(G):
                nmm_v[r, pl.ds(c * L, L)] = nmc

        def step(k, st):
            xp, yp, zp, xo, yo, zo, md2, done = st
            m127 = jnp.where(k == 0, _NEG_INF, jnp.float32(0.0))
            bestv = [neginf_v] * G
            besti = [jnp.full((L,), _BIG, jnp.int32)] * G
            for c in range(NCH):
                cx = uavp2_v[pl.ds(c * L, L)]
                cy = uavp2_v[pl.ds(SIZE + c * L, L)]
                cz = uavp2_v[pl.ds(2 * SIZE + c * L, L)]
                for r in range(G):
                    s = (nmm_v[r, pl.ds(c * L, L)]
                         + cx * xp[r] + cy * yp[r] + cz * zp[r])
                    if c == NCH - 1:
                        sd = (dnorm[r] + dpx2[r] * xp[r] + dpy2[r] * yp[r]
                              + dpz2[r] * zp[r]) + m127
                        s = jnp.where(lane15, sd, s)
                    gt = s > bestv[r]
                    bestv[r] = jnp.where(gt, s, bestv[r])
                    besti[r] = jnp.where(gt, iota + (c * L), besti[r])
            nxp, nyp, nzp = list(xp), list(yp), list(zp)
            nxo, nyo, nzo = list(xo), list(yo), list(zo)
            nmd, ndone = list(md2), list(done)
            for r in range(G):
                mval = jnp.max(bestv[r])
                idx = jnp.min(jnp.where(bestv[r] == mval, besti[r], _BIG))
                idxv = jnp.full((L,), idx, jnp.int32)
                is_dst = idxv == (SIZE - 1)
                nxp[r] = plsc.load_gather(uavp_v, [idxv])
                nyp[r] = plsc.load_gather(uavp_v, [idxv + SIZE])
                nzp[r] = plsc.load_gather(uavp_v, [idxv + 2 * SIZE])
                hx = plsc.load_gather(uav_v, [idxv])
                hy = plsc.load_gather(uav_v, [idxv + SIZE])
                hz = plsc.load_gather(uav_v, [idxv + 2 * SIZE])
                nxo[r] = jnp.where(is_dst, dox[r], hx)
                nyo[r] = jnp.where(is_dst, doy[r], hy)
                nzo[r] = jnp.where(is_dst, doz[r], hz)
                ex = nxo[r] - xo[r]
                ey = nyo[r] - yo[r]
                ez = nzo[r] - zo[r]
                dd2 = ex * ex + ey * ey + ez * ez
                nmd[r] = jnp.where(done[r], md2[r],
                                   jnp.maximum(md2[r], dd2))
                ndone[r] = done[r] | is_dst
                plsc.store_scatter(nmm_v.at[r], [idxv], neginf_v,
                                   mask=lane0 & (~is_dst))
            return (tuple(nxp), tuple(nyp), tuple(nzp),
                    tuple(nxo), tuple(nyo), tuple(nzo),
                    tuple(nmd), tuple(ndone))

        init = (tuple(px0), tuple(py0), tuple(pz0),
                tuple(ox0), tuple(oy0), tuple(oz0),
                tuple(zero_v for _ in range(G)),
                tuple(jnp.zeros((L,), jnp.bool_) for _ in range(G)))
        st = lax.fori_loop(0, SIZE, step, init)
        md2f = st[6]
        for r in range(G):
            plsc.store_scatter(md2_v, [rowv[r]], md2f[r], mask=lane0)
        return 0

    lax.fori_loop(0, NG, group_body, 0)
    pltpu.sync_copy(md2_v, out_hbm.at[pl.ds(base_row, RPW)])


_sc_path = functools.partial(
    pl.kernel,
    out_type=jax.ShapeDtypeStruct((N,), jnp.float32),
    mesh=plsc.VectorSubcoreMesh(core_axis_name="c", subcore_axis_name="s"),
    compiler_params=pltpu.CompilerParams(needs_layout_passes=False),
    scratch_types=[
        pltpu.VMEM((9 * L,), jnp.float32),
        pltpu.VMEM((3 * SIZE,), jnp.float32),
        pltpu.VMEM((3 * SIZE,), jnp.float32),
        pltpu.VMEM((3 * SIZE,), jnp.float32),
        pltpu.VMEM((SIZE,), jnp.float32),
        pltpu.VMEM((3 * RPW,), jnp.float32),
        pltpu.VMEM((3 * RPW,), jnp.float32),
        pltpu.VMEM((3 * RPW,), jnp.float32),
        pltpu.VMEM((3 * RPW,), jnp.float32),
        pltpu.VMEM((G, SIZE), jnp.float32),
        pltpu.VMEM((RPW,), jnp.float32),
    ],
)(_sc_body)


def _mean_sqrt_body(x_ref, o_ref):
    o_ref[0, 0] = jnp.sum(jnp.sqrt(x_ref[...])) * jnp.float32(1.0 / N)


_mean_sqrt = pl.pallas_call(
    _mean_sqrt_body,
    out_shape=jax.ShapeDtypeStruct((1, 1), jnp.float32),
    out_specs=pl.BlockSpec(memory_space=pltpu.SMEM),
)


def kernel(outputs, W):
    src = outputs[:N]
    dst = outputs[N:2 * N]
    uav = outputs[2 * N:]
    srco = src.T.reshape(-1)
    dsto = dst.T.reshape(-1)
    uavo = jnp.zeros((3, SIZE), jnp.float32).at[:, 1:SIZE - 1].set(uav.T).reshape(-1)
    wflat = jnp.repeat(W.reshape(-1), L)
    md2 = _sc_path(srco, dsto, uavo, wflat)
    return _mean_sqrt(md2.reshape(N // 128, 128))[0, 0]


# speedup vs baseline: 20.9980x; 1.4010x over previous
import functools

import jax
import jax.numpy as jnp
from jax import lax
from jax.experimental import pallas as pl
from jax.experimental.pallas import tpu as pltpu
from jax.experimental.pallas import tpu_sc as plsc

N = 8192
M = 126
SIZE = M + 2
NC, NS, L = 2, 16, 16
NW = NC * NS
RPW = N // NW
G = 8
NG = RPW // G
NCH = SIZE // L

_NEG_INF = float("-inf")
_BIG = 1 << 30


def _sc_body(srco, dsto, uavo, wflat, out_hbm,
             w_v, uav_v, uavp_v, uavp2_v, nm_v,
             srco_v, srcp_v, dsto_v, dstp_v,
             nmm_v, trace_v, md2_v):
    wid = lax.axis_index("s") * NC + lax.axis_index("c")
    base_row = wid * RPW

    pltpu.sync_copy(wflat, w_v)
    pltpu.sync_copy(uavo, uav_v)
    for coord in range(3):
        pltpu.sync_copy(srco.at[pl.ds(coord * N + base_row, RPW)],
                        srco_v.at[pl.ds(coord * RPW, RPW)])
        pltpu.sync_copy(dsto.at[pl.ds(coord * N + base_row, RPW)],
                        dsto_v.at[pl.ds(coord * RPW, RPW)])

    iota = lax.iota(jnp.int32, L)
    w = [[w_v[pl.ds((3 * i + j) * L, L)] for j in range(3)] for i in range(3)]

    neginf_v = jnp.full((L,), _NEG_INF, jnp.float32)
    zero_v = jnp.zeros((L,), jnp.float32)
    lane15 = iota == (L - 1)
    lane0 = iota == 0
    lane_lo = iota < G

    for c in range(NCH):
        ux = uav_v[pl.ds(c * L, L)]
        uy = uav_v[pl.ds(SIZE + c * L, L)]
        uz = uav_v[pl.ds(2 * SIZE + c * L, L)]
        pj = [ux * w[0][j] + uy * w[1][j] + uz * w[2][j] for j in range(3)]
        for j in range(3):
            uavp_v[pl.ds(j * SIZE + c * L, L)] = pj[j]
            uavp2_v[pl.ds(j * SIZE + c * L, L)] = pj[j] + pj[j]
        nm = -(pj[0] * pj[0] + pj[1] * pj[1] + pj[2] * pj[2])
        if c == 0:
            nm = jnp.where(lane0, _NEG_INF, nm)
        nm_v[pl.ds(c * L, L)] = nm
    for c in range(RPW // L):
        sx = srco_v[pl.ds(c * L, L)]
        sy = srco_v[pl.ds(RPW + c * L, L)]
        sz = srco_v[pl.ds(2 * RPW + c * L, L)]
        dx = dsto_v[pl.ds(c * L, L)]
        dy = dsto_v[pl.ds(RPW + c * L, L)]
        dz = dsto_v[pl.ds(2 * RPW + c * L, L)]
        for j in range(3):
            srcp_v[pl.ds(j * RPW + c * L, L)] = (
                sx * w[0][j] + sy * w[1][j] + sz * w[2][j])
            dstp_v[pl.ds(j * RPW + c * L, L)] = (
                dx * w[0][j] + dy * w[1][j] + dz * w[2][j])

    def group_body(g, _):
        rows = [g * G + r for r in range(G)]
        rowv = [jnp.full((L,), rows[r], jnp.int32) for r in range(G)]
        dgx = [plsc.load_gather(dstp_v, [rowv[r]]) for r in range(G)]
        dgy = [plsc.load_gather(dstp_v, [rowv[r] + RPW]) for r in range(G)]
        dgz = [plsc.load_gather(dstp_v, [rowv[r] + 2 * RPW]) for r in range(G)]
        dpx2 = [dgx[r] + dgx[r] for r in range(G)]
        dpy2 = [dgy[r] + dgy[r] for r in range(G)]
        dpz2 = [dgz[r] + dgz[r] for r in range(G)]
        dnorm = [-(dgx[r] * dgx[r] + dgy[r] * dgy[r] + dgz[r] * dgz[r])
                 for r in range(G)]
        px0 = [plsc.load_gather(srcp_v, [rowv[r]]) for r in range(G)]
        py0 = [plsc.load_gather(srcp_v, [rowv[r] + RPW]) for r in range(G)]
        pz0 = [plsc.load_gather(srcp_v, [rowv[r] + 2 * RPW]) for r in range(G)]
        for c in range(NCH):
            nmc = nm_v[pl.ds(c * L, L)]
            for r in range(G):
                nmm_v[r, pl.ds(c * L, L)] = nmc

        def step(k, st):
            xp, yp, zp = st
            m127 = jnp.where(k == 0, _NEG_INF, jnp.float32(0.0))
            kv = jnp.full((L,), k * L, jnp.int32)
            bestv = [neginf_v] * G
            besti = [jnp.full((L,), _BIG, jnp.int32)] * G
            for c in range(NCH):
                cx = uavp2_v[pl.ds(c * L, L)]
                cy = uavp2_v[pl.ds(SIZE + c * L, L)]
                cz = uavp2_v[pl.ds(2 * SIZE + c * L, L)]
                for r in range(G):
                    s = (nmm_v[r, pl.ds(c * L, L)]
                         + cx * xp[r] + cy * yp[r] + cz * zp[r])
                    if c == NCH - 1:
                        sd = (dnorm[r] + dpx2[r] * xp[r] + dpy2[r] * yp[r]
                              + dpz2[r] * zp[r]) + m127
                        s = jnp.where(lane15, sd, s)
                    gt = s > bestv[r]
                    bestv[r] = jnp.where(gt, s, bestv[r])
                    besti[r] = jnp.where(gt, iota + (c * L), besti[r])
            nxp, nyp, nzp = list(xp), list(yp), list(zp)
            for r in range(G):
                mval = jnp.max(bestv[r])
                idx = jnp.min(jnp.where(bestv[r] == mval, besti[r], _BIG))
                idxv = jnp.full((L,), idx, jnp.int32)
                nxp[r] = plsc.load_gather(uavp_v, [idxv])
                nyp[r] = plsc.load_gather(uavp_v, [idxv + SIZE])
                nzp[r] = plsc.load_gather(uavp_v, [idxv + 2 * SIZE])
                plsc.store_scatter(trace_v, [kv + r], idxv, mask=lane0)
                plsc.store_scatter(nmm_v.at[r], [idxv], neginf_v,
                                   mask=lane0 & (idxv != SIZE - 1))
            return (tuple(nxp), tuple(nyp), tuple(nzp))

        lax.fori_loop(0, SIZE, step,
                      (tuple(px0), tuple(py0), tuple(pz0)))

        rlane = jnp.int32(g * G) + jnp.where(lane_lo, iota, 0)
        dox = plsc.load_gather(dsto_v, [rlane])
        doy = plsc.load_gather(dsto_v, [rlane + RPW])
        doz = plsc.load_gather(dsto_v, [rlane + 2 * RPW])
        ox0 = plsc.load_gather(srco_v, [rlane])
        oy0 = plsc.load_gather(srco_v, [rlane + RPW])
        oz0 = plsc.load_gather(srco_v, [rlane + 2 * RPW])

        def replay(k, st):
            xo, yo, zo, md2, done = st
            idxk = trace_v[pl.ds(k * L, L)]
            is_dst = idxk == (SIZE - 1)
            hx = plsc.load_gather(uav_v, [idxk], mask=lane_lo)
            hy = plsc.load_gather(uav_v, [idxk + SIZE], mask=lane_lo)
            hz = plsc.load_gather(uav_v, [idxk + 2 * SIZE], mask=lane_lo)
            nxo = jnp.where(is_dst, dox, hx)
            nyo = jnp.where(is_dst, doy, hy)
            nzo = jnp.where(is_dst, doz, hz)
            ex = nxo - xo
            ey = nyo - yo
            ez = nzo - zo
            dd2 = ex * ex + ey * ey + ez * ez
            nmd = jnp.where(done, md2, jnp.maximum(md2, dd2))
            return (nxo, nyo, nzo, nmd, done | is_dst)

        st2 = lax.fori_loop(0, SIZE, replay,
                            (ox0, oy0, oz0, zero_v,
                             jnp.zeros((L,), jnp.bool_)))
        plsc.store_scatter(md2_v, [rlane], st2[3], mask=lane_lo)
        return 0

    lax.fori_loop(0, NG, group_body, 0)
    pltpu.sync_copy(md2_v, out_hbm.at[pl.ds(base_row, RPW)])


_sc_path = functools.partial(
    pl.kernel,
    out_type=jax.ShapeDtypeStruct((N,), jnp.float32),
    mesh=plsc.VectorSubcoreMesh(core_axis_name="c", subcore_axis_name="s"),
    compiler_params=pltpu.CompilerParams(needs_layout_passes=False),
    scratch_types=[
        pltpu.VMEM((9 * L,), jnp.float32),
        pltpu.VMEM((3 * SIZE,), jnp.float32),
        pltpu.VMEM((3 * SIZE,), jnp.float32),
        pltpu.VMEM((3 * SIZE,), jnp.float32),
        pltpu.VMEM((SIZE,), jnp.float32),
        pltpu.VMEM((3 * RPW,), jnp.float32),
        pltpu.VMEM((3 * RPW,), jnp.float32),
        pltpu.VMEM((3 * RPW,), jnp.float32),
        pltpu.VMEM((3 * RPW,), jnp.float32),
        pltpu.VMEM((G, SIZE), jnp.float32),
        pltpu.VMEM((SIZE * L,), jnp.int32),
        pltpu.VMEM((RPW,), jnp.float32),
    ],
)(_sc_body)


def _mean_sqrt_body(x_ref, o_ref):
    o_ref[0, 0] = jnp.sum(jnp.sqrt(x_ref[...])) * jnp.float32(1.0 / N)


_mean_sqrt = pl.pallas_call(
    _mean_sqrt_body,
    out_shape=jax.ShapeDtypeStruct((1, 1), jnp.float32),
    out_specs=pl.BlockSpec(memory_space=pltpu.SMEM),
)


def kernel(outputs, W):
    src = outputs[:N]
    dst = outputs[N:2 * N]
    uav = outputs[2 * N:]
    srco = src.T.reshape(-1)
    dsto = dst.T.reshape(-1)
    uavo = jnp.zeros((3, SIZE), jnp.float32).at[:, 1:SIZE - 1].set(uav.T).reshape(-1)
    wflat = jnp.repeat(W.reshape(-1), L)
    md2 = _sc_path(srco, dsto, uavo, wflat)
    return _mean_sqrt(md2.reshape(N // 128, 128))[0, 0]


# speedup vs baseline: 22.3669x; 1.0652x over previous
import functools

import jax
import jax.numpy as jnp
from jax import lax
from jax.experimental import pallas as pl
from jax.experimental.pallas import tpu as pltpu
from jax.experimental.pallas import tpu_sc as plsc

N = 8192
M = 126
SIZE = M + 2
NC, NS, L = 2, 16, 16
NW = NC * NS
RPW = N // NW
G = 8
NG = RPW // G
NCH = SIZE // L

_NEG_INF = float("-inf")
_BIG = 1 << 30


def _sc_body(srco, dsto, uavo, wflat, out_hbm,
             w_v, uav_v, uavp_v, uavp2_v, nm_v,
             srco_v, srcp_v, dsto_v, dstp_v,
             nmm_v, trace_v, md2_v):
    wid = lax.axis_index("s") * NC + lax.axis_index("c")
    base_row = wid * RPW

    pltpu.sync_copy(wflat, w_v)
    pltpu.sync_copy(uavo, uav_v)
    for coord in range(3):
        pltpu.sync_copy(srco.at[pl.ds(coord * N + base_row, RPW)],
                        srco_v.at[pl.ds(coord * RPW, RPW)])
        pltpu.sync_copy(dsto.at[pl.ds(coord * N + base_row, RPW)],
                        dsto_v.at[pl.ds(coord * RPW, RPW)])

    iota = lax.iota(jnp.int32, L)
    w = [[w_v[pl.ds((3 * i + j) * L, L)] for j in range(3)] for i in range(3)]

    neginf_v = jnp.full((L,), _NEG_INF, jnp.float32)
    zero_v = jnp.zeros((L,), jnp.float32)
    lane15 = iota == (L - 1)
    lane0 = iota == 0
    lane_lo = iota < G

    for c in range(NCH):
        ux = uav_v[pl.ds(c * L, L)]
        uy = uav_v[pl.ds(SIZE + c * L, L)]
        uz = uav_v[pl.ds(2 * SIZE + c * L, L)]
        pj = [ux * w[0][j] + uy * w[1][j] + uz * w[2][j] for j in range(3)]
        for j in range(3):
            uavp_v[pl.ds(j * SIZE + c * L, L)] = pj[j]
            uavp2_v[pl.ds(j * SIZE + c * L, L)] = pj[j] + pj[j]
        nm = -(pj[0] * pj[0] + pj[1] * pj[1] + pj[2] * pj[2])
        if c == 0:
            nm = jnp.where(lane0, _NEG_INF, nm)
        nm_v[pl.ds(c * L, L)] = nm
    for c in range(RPW // L):
        sx = srco_v[pl.ds(c * L, L)]
        sy = srco_v[pl.ds(RPW + c * L, L)]
        sz = srco_v[pl.ds(2 * RPW + c * L, L)]
        dx = dsto_v[pl.ds(c * L, L)]
        dy = dsto_v[pl.ds(RPW + c * L, L)]
        dz = dsto_v[pl.ds(2 * RPW + c * L, L)]
        for j in range(3):
            srcp_v[pl.ds(j * RPW + c * L, L)] = (
                sx * w[0][j] + sy * w[1][j] + sz * w[2][j])
            dstp_v[pl.ds(j * RPW + c * L, L)] = (
                dx * w[0][j] + dy * w[1][j] + dz * w[2][j])

    def group_body(g, _):
        rows = [g * G + r for r in range(G)]
        rowv = [jnp.full((L,), rows[r], jnp.int32) for r in range(G)]
        dgx = [plsc.load_gather(dstp_v, [rowv[r]]) for r in range(G)]
        dgy = [plsc.load_gather(dstp_v, [rowv[r] + RPW]) for r in range(G)]
        dgz = [plsc.load_gather(dstp_v, [rowv[r] + 2 * RPW]) for r in range(G)]
        dpx2 = [dgx[r] + dgx[r] for r in range(G)]
        dpy2 = [dgy[r] + dgy[r] for r in range(G)]
        dpz2 = [dgz[r] + dgz[r] for r in range(G)]
        dnorm = [-(dgx[r] * dgx[r] + dgy[r] * dgy[r] + dgz[r] * dgz[r])
                 for r in range(G)]
        px0 = [plsc.load_gather(srcp_v, [rowv[r]]) for r in range(G)]
        py0 = [plsc.load_gather(srcp_v, [rowv[r] + RPW]) for r in range(G)]
        pz0 = [plsc.load_gather(srcp_v, [rowv[r] + 2 * RPW]) for r in range(G)]
        for c in range(NCH):
            nmc = nm_v[pl.ds(c * L, L)]
            for r in range(G):
                nmm_v[r, pl.ds(c * L, L)] = nmc

        def step(k, st):
            xp, yp, zp = st
            m127 = jnp.where(k == 0, _NEG_INF, jnp.float32(0.0))
            kv = jnp.full((L,), k * L, jnp.int32)
            bestv = [neginf_v] * G
            besti = [jnp.full((L,), _BIG, jnp.int32)] * G
            for c in range(NCH):
                cx = uavp2_v[pl.ds(c * L, L)]
                cy = uavp2_v[pl.ds(SIZE + c * L, L)]
                cz = uavp2_v[pl.ds(2 * SIZE + c * L, L)]
                for r in range(G):
                    s = (nmm_v[r, pl.ds(c * L, L)]
                         + cx * xp[r] + cy * yp[r] + cz * zp[r])
                    if c == NCH - 1:
                        sd = (dnorm[r] + dpx2[r] * xp[r] + dpy2[r] * yp[r]
                              + dpz2[r] * zp[r]) + m127
                        s = jnp.where(lane15, sd, s)
                    gt = s > bestv[r]
                    bestv[r] = jnp.maximum(s, bestv[r])
                    besti[r] = jnp.where(gt, iota + (c * L), besti[r])
            nxp, nyp, nzp = list(xp), list(yp), list(zp)
            for r in range(G):
                mval = jnp.max(bestv[r])
                idx = jnp.min(jnp.where(bestv[r] == mval, besti[r], _BIG))
                idxv = jnp.full((L,), idx, jnp.int32)
                nxp[r] = plsc.load_gather(uavp_v, [idxv])
                nyp[r] = plsc.load_gather(uavp_v, [idxv + SIZE])
                nzp[r] = plsc.load_gather(uavp_v, [idxv + 2 * SIZE])
                plsc.store_scatter(trace_v, [kv + r], idxv, mask=lane0)
                plsc.store_scatter(nmm_v.at[r], [idxv], neginf_v,
                                   mask=lane0 & (idxv != SIZE - 1))
            return (tuple(nxp), tuple(nyp), tuple(nzp))

        lax.fori_loop(0, SIZE, step,
                      (tuple(px0), tuple(py0), tuple(pz0)))

        rlane = jnp.int32(g * G) + jnp.where(lane_lo, iota, 0)
        dox = plsc.load_gather(dsto_v, [rlane])
        doy = plsc.load_gather(dsto_v, [rlane + RPW])
        doz = plsc.load_gather(dsto_v, [rlane + 2 * RPW])
        ox0 = plsc.load_gather(srco_v, [rlane])
        oy0 = plsc.load_gather(srco_v, [rlane + RPW])
        oz0 = plsc.load_gather(srco_v, [rlane + 2 * RPW])

        def replay(k, st):
            xo, yo, zo, md2, done = st
            idxk = trace_v[pl.ds(k * L, L)]
            is_dst = idxk == (SIZE - 1)
            hx = plsc.load_gather(uav_v, [idxk], mask=lane_lo)
            hy = plsc.load_gather(uav_v, [idxk + SIZE], mask=lane_lo)
            hz = plsc.load_gather(uav_v, [idxk + 2 * SIZE], mask=lane_lo)
            nxo = jnp.where(is_dst, dox, hx)
            nyo = jnp.where(is_dst, doy, hy)
            nzo = jnp.where(is_dst, doz, hz)
            ex = nxo - xo
            ey = nyo - yo
            ez = nzo - zo
            dd2 = ex * ex + ey * ey + ez * ez
            nmd = jnp.where(done, md2, jnp.maximum(md2, dd2))
            return (nxo, nyo, nzo, nmd, done | is_dst)

        st2 = lax.fori_loop(0, SIZE, replay,
                            (ox0, oy0, oz0, zero_v,
                             jnp.zeros((L,), jnp.bool_)))
        plsc.store_scatter(md2_v, [rlane], st2[3], mask=lane_lo)
        return 0

    lax.fori_loop(0, NG, group_body, 0)
    pltpu.sync_copy(md2_v, out_hbm.at[pl.ds(base_row, RPW)])


_sc_path = functools.partial(
    pl.kernel,
    out_type=jax.ShapeDtypeStruct((N,), jnp.float32),
    mesh=plsc.VectorSubcoreMesh(core_axis_name="c", subcore_axis_name="s"),
    compiler_params=pltpu.CompilerParams(needs_layout_passes=False),
    scratch_types=[
        pltpu.VMEM((9 * L,), jnp.float32),
        pltpu.VMEM((3 * SIZE,), jnp.float32),
        pltpu.VMEM((3 * SIZE,), jnp.float32),
        pltpu.VMEM((3 * SIZE,), jnp.float32),
        pltpu.VMEM((SIZE,), jnp.float32),
        pltpu.VMEM((3 * RPW,), jnp.float32),
        pltpu.VMEM((3 * RPW,), jnp.float32),
        pltpu.VMEM((3 * RPW,), jnp.float32),
        pltpu.VMEM((3 * RPW,), jnp.float32),
        pltpu.VMEM((G, SIZE), jnp.float32),
        pltpu.VMEM((SIZE * L,), jnp.int32),
        pltpu.VMEM((RPW,), jnp.float32),
    ],
)(_sc_body)


def _mean_sqrt_body(x_ref, o_ref):
    o_ref[0, 0] = jnp.sum(jnp.sqrt(x_ref[...])) * jnp.float32(1.0 / N)


_mean_sqrt = pl.pallas_call(
    _mean_sqrt_body,
    out_shape=jax.ShapeDtypeStruct((1, 1), jnp.float32),
    out_specs=pl.BlockSpec(memory_space=pltpu.SMEM),
)


def kernel(outputs, W):
    src = outputs[:N]
    dst = outputs[N:2 * N]
    uav = outputs[2 * N:]
    srco = src.T.reshape(-1)
    dsto = dst.T.reshape(-1)
    uavo = jnp.zeros((3, SIZE), jnp.float32).at[:, 1:SIZE - 1].set(uav.T).reshape(-1)
    wflat = jnp.repeat(W.reshape(-1), L)
    md2 = _sc_path(srco, dsto, uavo, wflat)
    return _mean_sqrt(md2.reshape(N // 128, 128))[0, 0]


# speedup vs baseline: 25.9844x; 1.1617x over previous
import functools

import jax
import jax.numpy as jnp
from jax import lax
from jax.experimental import pallas as pl
from jax.experimental.pallas import tpu as pltpu
from jax.experimental.pallas import tpu_sc as plsc

N = 8192
M = 126
SIZE = M + 2
NC, NS, L = 2, 16, 16
NW = NC * NS
RPW = N // NW
G = 8
NG = RPW // G
NCH = SIZE // L

_NEG_INF = float("-inf")
_BIG = 1 << 30


def _sc_body(srco, dsto, uavo, wflat, out_hbm,
             w_v, uav_v, uavp_v, uavp2_v, nm_v,
             srco_v, srcp_v, dsto_v, dstp_v,
             nmm_v, trace_v, md2_v, dstsc_v):
    wid = lax.axis_index("s") * NC + lax.axis_index("c")
    base_row = wid * RPW

    pltpu.sync_copy(wflat, w_v)
    pltpu.sync_copy(uavo, uav_v)
    for coord in range(3):
        pltpu.sync_copy(srco.at[pl.ds(coord * N + base_row, RPW)],
                        srco_v.at[pl.ds(coord * RPW, RPW)])
        pltpu.sync_copy(dsto.at[pl.ds(coord * N + base_row, RPW)],
                        dsto_v.at[pl.ds(coord * RPW, RPW)])

    iota = lax.iota(jnp.int32, L)
    w = [[w_v[pl.ds((3 * i + j) * L, L)] for j in range(3)] for i in range(3)]

    neginf_v = jnp.full((L,), _NEG_INF, jnp.float32)
    zero_v = jnp.zeros((L,), jnp.float32)
    lane15 = iota == (L - 1)
    lane0 = iota == 0
    lane_lo = iota < G

    for c in range(NCH):
        ux = uav_v[pl.ds(c * L, L)]
        uy = uav_v[pl.ds(SIZE + c * L, L)]
        uz = uav_v[pl.ds(2 * SIZE + c * L, L)]
        pj = [ux * w[0][j] + uy * w[1][j] + uz * w[2][j] for j in range(3)]
        for j in range(3):
            uavp_v[pl.ds(j * SIZE + c * L, L)] = pj[j]
            uavp2_v[pl.ds(j * SIZE + c * L, L)] = pj[j] + pj[j]
        nm = -(pj[0] * pj[0] + pj[1] * pj[1] + pj[2] * pj[2])
        if c == 0:
            nm = jnp.where(lane0, _NEG_INF, nm)
        nm_v[pl.ds(c * L, L)] = nm
    for c in range(RPW // L):
        sx = srco_v[pl.ds(c * L, L)]
        sy = srco_v[pl.ds(RPW + c * L, L)]
        sz = srco_v[pl.ds(2 * RPW + c * L, L)]
        dx = dsto_v[pl.ds(c * L, L)]
        dy = dsto_v[pl.ds(RPW + c * L, L)]
        dz = dsto_v[pl.ds(2 * RPW + c * L, L)]
        for j in range(3):
            srcp_v[pl.ds(j * RPW + c * L, L)] = (
                sx * w[0][j] + sy * w[1][j] + sz * w[2][j])
            dstp_v[pl.ds(j * RPW + c * L, L)] = (
                dx * w[0][j] + dy * w[1][j] + dz * w[2][j])

    def group_body(g, _):
        rows = [g * G + r for r in range(G)]
        rowv = [jnp.full((L,), rows[r], jnp.int32) for r in range(G)]
        dgx = [plsc.load_gather(dstp_v, [rowv[r]]) for r in range(G)]
        dgy = [plsc.load_gather(dstp_v, [rowv[r] + RPW]) for r in range(G)]
        dgz = [plsc.load_gather(dstp_v, [rowv[r] + 2 * RPW]) for r in range(G)]
        for r in range(G):
            dstsc_v[0, r] = dgx[r] + dgx[r]
            dstsc_v[1, r] = dgy[r] + dgy[r]
            dstsc_v[2, r] = dgz[r] + dgz[r]
            dstsc_v[3, r] = -(dgx[r] * dgx[r] + dgy[r] * dgy[r]
                              + dgz[r] * dgz[r])
        px0 = [plsc.load_gather(srcp_v, [rowv[r]]) for r in range(G)]
        py0 = [plsc.load_gather(srcp_v, [rowv[r] + RPW]) for r in range(G)]
        pz0 = [plsc.load_gather(srcp_v, [rowv[r] + 2 * RPW]) for r in range(G)]
        for c in range(NCH):
            nmc = nm_v[pl.ds(c * L, L)]
            for r in range(G):
                nmm_v[r, pl.ds(c * L, L)] = nmc

        def make_step(first):
            def step(k, st):
                xp, yp, zp = st
                kv = jnp.full((L,), k * L, jnp.int32)
                bestv = [None] * G
                besti = [None] * G
                for c in range(NCH):
                    cx = uavp2_v[pl.ds(c * L, L)]
                    cy = uavp2_v[pl.ds(SIZE + c * L, L)]
                    cz = uavp2_v[pl.ds(2 * SIZE + c * L, L)]
                    for r in range(G):
                        s = (nmm_v[r, pl.ds(c * L, L)]
                             + cx * xp[r] + cy * yp[r] + cz * zp[r])
                        if c == NCH - 1:
                            if first:
                                s = jnp.where(lane15, neginf_v, s)
                            else:
                                sd = (dstsc_v[3, r] + dstsc_v[0, r] * xp[r]
                                      + dstsc_v[1, r] * yp[r]
                                      + dstsc_v[2, r] * zp[r])
                                s = jnp.where(lane15, sd, s)
                        if c == 0:
                            bestv[r] = s
                            besti[r] = jnp.int32(0)
                        else:
                            gt = s > bestv[r]
                            bestv[r] = jnp.maximum(s, bestv[r])
                            besti[r] = jnp.where(gt, jnp.int32(c), besti[r])
                nxp, nyp, nzp = list(xp), list(yp), list(zp)
                for r in range(G):
                    mval = jnp.max(bestv[r])
                    gidx = besti[r] * L + iota
                    idx = jnp.min(jnp.where(bestv[r] == mval, gidx, _BIG))
                    idxv = jnp.full((L,), idx, jnp.int32)
                    nxp[r] = plsc.load_gather(uavp_v, [idxv])
                    nyp[r] = plsc.load_gather(uavp_v, [idxv + SIZE])
                    nzp[r] = plsc.load_gather(uavp_v, [idxv + 2 * SIZE])
                    plsc.store_scatter(trace_v, [kv + r], idxv, mask=lane0)
                    plsc.store_scatter(nmm_v.at[r], [idxv], neginf_v,
                                       mask=lane0 & (idxv != SIZE - 1))
                return (tuple(nxp), tuple(nyp), tuple(nzp))
            return step

        st0 = make_step(True)(0, (tuple(px0), tuple(py0), tuple(pz0)))
        lax.fori_loop(1, SIZE, make_step(False), st0)

        rlane = jnp.int32(g * G) + jnp.where(lane_lo, iota, 0)
        dox = plsc.load_gather(dsto_v, [rlane])
        doy = plsc.load_gather(dsto_v, [rlane + RPW])
        doz = plsc.load_gather(dsto_v, [rlane + 2 * RPW])
        ox0 = plsc.load_gather(srco_v, [rlane])
        oy0 = plsc.load_gather(srco_v, [rlane + RPW])
        oz0 = plsc.load_gather(srco_v, [rlane + 2 * RPW])

        def replay(k, st):
            xo, yo, zo, md2, done = st
            idxk = trace_v[pl.ds(k * L, L)]
            is_dst = idxk == (SIZE - 1)
            hx = plsc.load_gather(uav_v, [idxk], mask=lane_lo)
            hy = plsc.load_gather(uav_v, [idxk + SIZE], mask=lane_lo)
            hz = plsc.load_gather(uav_v, [idxk + 2 * SIZE], mask=lane_lo)
            nxo = jnp.where(is_dst, dox, hx)
            nyo = jnp.where(is_dst, doy, hy)
            nzo = jnp.where(is_dst, doz, hz)
            ex = nxo - xo
            ey = nyo - yo
            ez = nzo - zo
            dd2 = ex * ex + ey * ey + ez * ez
            nmd = jnp.where(done, md2, jnp.maximum(md2, dd2))
            return (nxo, nyo, nzo, nmd, done | is_dst)

        st2 = lax.fori_loop(0, SIZE, replay,
                            (ox0, oy0, oz0, zero_v,
                             jnp.zeros((L,), jnp.bool_)))
        plsc.store_scatter(md2_v, [rlane], st2[3], mask=lane_lo)
        return 0

    lax.fori_loop(0, NG, group_body, 0)
    pltpu.sync_copy(md2_v, out_hbm.at[pl.ds(base_row, RPW)])


_sc_path = functools.partial(
    pl.kernel,
    out_type=jax.ShapeDtypeStruct((N,), jnp.float32),
    mesh=plsc.VectorSubcoreMesh(core_axis_name="c", subcore_axis_name="s"),
    compiler_params=pltpu.CompilerParams(needs_layout_passes=False),
    scratch_types=[
        pltpu.VMEM((9 * L,), jnp.float32),
        pltpu.VMEM((3 * SIZE,), jnp.float32),
        pltpu.VMEM((3 * SIZE,), jnp.float32),
        pltpu.VMEM((3 * SIZE,), jnp.float32),
        pltpu.VMEM((SIZE,), jnp.float32),
        pltpu.VMEM((3 * RPW,), jnp.float32),
        pltpu.VMEM((3 * RPW,), jnp.float32),
        pltpu.VMEM((3 * RPW,), jnp.float32),
        pltpu.VMEM((3 * RPW,), jnp.float32),
        pltpu.VMEM((G, SIZE), jnp.float32),
        pltpu.VMEM((SIZE * L,), jnp.int32),
        pltpu.VMEM((RPW,), jnp.float32),
        pltpu.VMEM((4, G, L), jnp.float32),
    ],
)(_sc_body)


def _mean_sqrt_body(x_ref, o_ref):
    o_ref[0, 0] = jnp.sum(jnp.sqrt(x_ref[...])) * jnp.float32(1.0 / N)


_mean_sqrt = pl.pallas_call(
    _mean_sqrt_body,
    out_shape=jax.ShapeDtypeStruct((1, 1), jnp.float32),
    out_specs=pl.BlockSpec(memory_space=pltpu.SMEM),
)


def kernel(outputs, W):
    src = outputs[:N]
    dst = outputs[N:2 * N]
    uav = outputs[2 * N:]
    srco = src.T.reshape(-1)
    dsto = dst.T.reshape(-1)
    uavo = jnp.zeros((3, SIZE), jnp.float32).at[:, 1:SIZE - 1].set(uav.T).reshape(-1)
    wflat = jnp.repeat(W.reshape(-1), L)
    md2 = _sc_path(srco, dsto, uavo, wflat)
    return _mean_sqrt(md2.reshape(N // 128, 128))[0, 0]
